# baseline (speedup 1.0000x reference)
import functools

import jax
import jax.numpy as jnp
from jax import lax
from jax.experimental import pallas as pl
from jax.experimental.pallas import tpu as pltpu
from jax.experimental.pallas import tpu_sc as plsc

_EPS = 1e-12


def _sc_gather(x_flat, midx, didx, tm3, tok_pad, maug, daug, kvt_flat, vrows):
    n_tok = x_flat.shape[0]
    dw = tok_pad.shape[1]
    l, l2, b = tm3.shape
    d = 64
    info = plsc.get_sparse_core_info()
    nc, ns = info.num_cores, info.num_subcores
    nw = nc * ns
    n_tokw = 8
    tok_pw = n_tok // n_tokw
    tch = 64
    n_tok_ch = tok_pw // tch
    total_units = 2 * l * l2
    per_other = total_units // nw + 2
    per_tok = (total_units - (nw - n_tokw) * per_other) // n_tokw
    assert per_tok * n_tokw + per_other * (nw - n_tokw) == total_units
    hd = d // 2
    nbg = b // 16
    mesh = plsc.VectorSubcoreMesh(core_axis_name="c", subcore_axis_name="s")

    @functools.partial(
        pl.kernel,
        out_type=(
            jax.ShapeDtypeStruct((n_tok, dw), jnp.float32),
            jax.ShapeDtypeStruct((l, l2, d, b), jnp.float32),
            jax.ShapeDtypeStruct((l, l2, d, b), jnp.float32),
        ),
        mesh=mesh,
        compiler_params=pltpu.CompilerParams(
            use_tc_tiling_on_sc=True, needs_layout_passes=False),
        scratch_types=[
            pltpu.VMEM((tok_pw,), jnp.int32),
            pltpu.VMEM((tok_pw,), jnp.int32),
            pltpu.VMEM((tok_pw,), jnp.int32),
            pltpu.VMEM((2 * d * vrows,), jnp.float32),
            pltpu.VMEM((b,), jnp.int32),
            pltpu.VMEM((hd, b), jnp.float32),
            pltpu.VMEM((hd, b), jnp.float32),
            [pltpu.VMEM((tch, dw), jnp.float32)] * 2,
            [pltpu.SemaphoreType.DMA] * 2,
            [pltpu.SemaphoreType.DMA] * 2,
        ],
    )
    def k(x_hbm, m_hbm, d_hbm, tm_hbm, tok_hbm, maug_hbm, daug_hbm, kvt_hbm,
          h_out, k_out, v_out,
          xi_v, mi_v, di_v, kvt_v, idx_v, kst, vst, hbufs, gsems, wsems):
        wid = lax.axis_index("s") * nc + lax.axis_index("c")

        def wait_write(buf, dst, sem):
            pltpu.make_async_copy(buf, dst, sem).wait()

        @pl.when(wid < n_tokw)
        def _():
            tbase = wid * tok_pw
            pltpu.sync_copy(x_hbm.at[pl.ds(tbase, tok_pw)], xi_v)
            pltpu.sync_copy(m_hbm.at[pl.ds(tbase, tok_pw)], mi_v)
            pltpu.sync_copy(d_hbm.at[pl.ds(tbase, tok_pw)], di_v)
            for c in range(n_tok_ch):
                i = c % 2
                off = c * tch
                if c >= 2:
                    wait_write(hbufs[i], h_out.at[pl.ds(0, tch)], wsems[i])
                pltpu.async_copy(tok_hbm.at[xi_v.at[pl.ds(off, tch)]], hbufs[i], gsems[i]).wait()
                a = pltpu.async_copy(maug_hbm.at[mi_v.at[pl.ds(off, tch)]], hbufs[i], gsems[i], add=True)
                bb = pltpu.async_copy(daug_hbm.at[di_v.at[pl.ds(off, tch)]], hbufs[i], gsems[i], add=True)
                a.wait()
                bb.wait()
                pltpu.async_copy(hbufs[i], h_out.at[pl.ds(tbase + off, tch)], wsems[i])
            for c in range(max(0, n_tok_ch - 2), n_tok_ch):
                wait_write(hbufs[c % 2], h_out.at[pl.ds(0, tch)], wsems[c % 2])

        pltpu.sync_copy(kvt_hbm, kvt_v)

        gs = 8

        def fill(st, tab):
            def _bg(bg, carry2):
                col = bg * 16
                idx16 = idx_v[pl.ds(col, 16)]

                def loads(g):
                    return [
                        plsc.load_gather(
                            tab.at[pl.ds((g * gs + e) * vrows, vrows)], [idx16])
                        for e in range(gs)
                    ]

                def stores(vals, g):
                    for e in range(gs):
                        st[g * gs + e, pl.ds(col, 16)] = vals[e]

                prev = loads(0)
                for g in range(1, hd // gs):
                    cur = loads(g)
                    stores(prev, g - 1)
                    prev = cur
                stores(prev, hd // gs - 1)
                return carry2

            lax.fori_loop(0, nbg, _bg, 0)

        base_u = jnp.where(wid < n_tokw, wid * per_tok,
                           n_tokw * per_tok + (wid - n_tokw) * per_other)
        n_u = jnp.where(wid < n_tokw, per_tok, per_other)

        def unit_body(t, carry):
            u = base_u + t
            slab = u // 2
            li = slab // l2
            ji = slab % l2
            d0 = (u % 2) * hd

            @pl.when(jnp.logical_or(t == 0, u % 2 == 0))
            def _():
                pltpu.sync_copy(tm_hbm.at[li, ji], idx_v)
            ktab = kvt_v.at[pl.ds(d0 * vrows, hd * vrows)]
            vtab = kvt_v.at[pl.ds((d + d0) * vrows, hd * vrows)]

            @pl.when(t > 0)
            def _():
                wait_write(kst, k_out.at[0, 0, pl.ds(0, hd)], wsems[0])

            fill(kst, ktab)
            pltpu.async_copy(kst, k_out.at[li, ji, pl.ds(d0, hd)], wsems[0])

            @pl.when(t > 0)
            def _():
                wait_write(vst, v_out.at[0, 0, pl.ds(0, hd)], wsems[1])

            fill(vst, vtab)
            pltpu.async_copy(vst, v_out.at[li, ji, pl.ds(d0, hd)], wsems[1])
            return carry

        lax.fori_loop(0, n_u, unit_body, 0)
        wait_write(kst, k_out.at[0, 0, pl.ds(0, hd)], wsems[0])
        wait_write(vst, v_out.at[0, 0, pl.ds(0, hd)], wsems[1])

    return k(x_flat, midx, didx, tm3, tok_pad, maug, daug, kvt_flat)


def _layernorm_tc(h, gamma, beta, l):
    n, dw = h.shape
    d = gamma.shape[0]
    blk = n // l

    def body(h_ref, g_ref, b_ref, o_ref):
        hv = h_ref[:, :d]
        u = jnp.mean(hv, axis=-1, keepdims=True)
        c = hv - u
        s = jnp.mean(c * c, axis=-1, keepdims=True)
        res = g_ref[...] * (c * lax.rsqrt(s + _EPS)) + b_ref[...]
        o_ref[0] = jnp.transpose(res, (1, 0))

    return pl.pallas_call(
        body,
        grid=(l,),
        in_specs=[
            pl.BlockSpec((blk, dw), lambda i: (i, 0)),
            pl.BlockSpec((1, d), lambda i: (0, 0)),
            pl.BlockSpec((1, d), lambda i: (0, 0)),
        ],
        out_specs=pl.BlockSpec((1, d, blk), lambda i: (i, 0, 0)),
        out_shape=jax.ShapeDtypeStruct((l, d, blk), jnp.float32),
    )(h, gamma.reshape(1, d), beta.reshape(1, d))


def kernel(x, stamp, time_matrix, tok_table, weekday_table, day_table, month_table, K_table, V_table, gamma, beta):
    b, l = x.shape
    d = tok_table.shape[1]
    dw = 2 * d

    x_flat = x.T.reshape(-1)
    tm3 = jnp.transpose(time_matrix, (1, 2, 0))
    vrows = 264
    kvt = jnp.concatenate([K_table.T, V_table.T], axis=0)
    kvt = jnp.pad(kvt, ((0, 0), (0, vrows - kvt.shape[1]))).reshape(-1)
    m_sent = month_table.shape[0]
    d_sent = day_table.shape[0]
    midx = jnp.concatenate(
        [jnp.full((1, b), m_sent, jnp.int32), stamp[:, :, 0].T], axis=0).reshape(-1)
    didx = jnp.concatenate(
        [jnp.full((1, b), d_sent, jnp.int32), stamp[:, :, 1].T], axis=0).reshape(-1)
    tok_pad = jnp.pad(tok_table, ((0, 0), (0, dw - d)))
    maug = jnp.pad(month_table, ((0, 3), (0, dw - d)))
    daug = jnp.pad(day_table, ((0, 8), (0, dw - d)))

    h, kout, vout = _sc_gather(x_flat, midx, didx, tm3, tok_pad,
                               maug, daug, kvt, vrows)
    out = _layernorm_tc(h, gamma, beta, l)
    kf = jnp.transpose(kout, (3, 0, 1, 2))
    vf = jnp.transpose(vout, (3, 0, 1, 2))
    return (jnp.transpose(out, (2, 0, 1)), kf, vf)


# speedup vs baseline: 1.3212x; 1.3212x over previous
import functools

import jax
import jax.numpy as jnp
from jax import lax
from jax.experimental import pallas as pl
from jax.experimental.pallas import tpu as pltpu
from jax.experimental.pallas import tpu_sc as plsc

_EPS = 1e-12


def _sc_gather(x_flat, midx, didx, tm3, tok_pad, maug, daug, kvt_flat, vrows):
    n_tok = x_flat.shape[0]
    dw = tok_pad.shape[1]
    l, l2, b = tm3.shape
    d = 64
    info = plsc.get_sparse_core_info()
    nc, ns = info.num_cores, info.num_subcores
    nw = nc * ns
    tok_pw = n_tok // nw
    tch = 64
    n_tok_ch = tok_pw // tch
    n_units = 2 * l * l2 // nw
    hd = d // 2
    nbg = b // 16
    mesh = plsc.VectorSubcoreMesh(core_axis_name="c", subcore_axis_name="s")

    @functools.partial(
        pl.kernel,
        out_type=(
            jax.ShapeDtypeStruct((n_tok, dw), jnp.float32),
            jax.ShapeDtypeStruct((l, l2, d, b), jnp.float32),
            jax.ShapeDtypeStruct((l, l2, d, b), jnp.float32),
        ),
        mesh=mesh,
        compiler_params=pltpu.CompilerParams(
            use_tc_tiling_on_sc=True, needs_layout_passes=False),
        scratch_types=[
            pltpu.VMEM((tok_pw,), jnp.int32),
            pltpu.VMEM((tok_pw,), jnp.int32),
            pltpu.VMEM((tok_pw,), jnp.int32),
            pltpu.VMEM((2 * d * vrows,), jnp.float32),
            pltpu.VMEM((b,), jnp.int32),
            pltpu.VMEM((hd, b), jnp.float32),
            pltpu.VMEM((hd, b), jnp.float32),
            [pltpu.VMEM((tch, dw), jnp.float32)] * 2,
            [pltpu.SemaphoreType.DMA] * 2,
            [pltpu.SemaphoreType.DMA] * 2,
        ],
    )
    def k(x_hbm, m_hbm, d_hbm, tm_hbm, tok_hbm, maug_hbm, daug_hbm, kvt_hbm,
          h_out, k_out, v_out,
          xi_v, mi_v, di_v, kvt_v, idx_v, kst, vst, hbufs, gsems, wsems):
        wid = lax.axis_index("s") * nc + lax.axis_index("c")
        tbase = wid * tok_pw
        pltpu.sync_copy(x_hbm.at[pl.ds(tbase, tok_pw)], xi_v)
        pltpu.sync_copy(m_hbm.at[pl.ds(tbase, tok_pw)], mi_v)
        pltpu.sync_copy(d_hbm.at[pl.ds(tbase, tok_pw)], di_v)

        def wait_write(buf, dst, sem):
            pltpu.make_async_copy(buf, dst, sem).wait()

        for c in range(n_tok_ch):
            i = c % 2
            off = c * tch
            if c >= 2:
                wait_write(hbufs[i], h_out.at[pl.ds(0, tch)], wsems[i])
            pltpu.async_copy(tok_hbm.at[xi_v.at[pl.ds(off, tch)]], hbufs[i], gsems[i]).wait()
            a = pltpu.async_copy(maug_hbm.at[mi_v.at[pl.ds(off, tch)]], hbufs[i], gsems[i], add=True)
            bb = pltpu.async_copy(daug_hbm.at[di_v.at[pl.ds(off, tch)]], hbufs[i], gsems[i], add=True)
            a.wait()
            bb.wait()
            pltpu.async_copy(hbufs[i], h_out.at[pl.ds(tbase + off, tch)], wsems[i])
        pltpu.sync_copy(kvt_hbm, kvt_v)
        for c in range(max(0, n_tok_ch - 2), n_tok_ch):
            wait_write(hbufs[c % 2], h_out.at[pl.ds(0, tch)], wsems[c % 2])

        gs = 8

        def fill(st, tab):
            def _bg(bg, carry2):
                col = bg * 16
                idx16 = idx_v[pl.ds(col, 16)]

                def loads(g):
                    return [
                        plsc.load_gather(
                            tab.at[pl.ds((g * gs + e) * vrows, vrows)], [idx16])
                        for e in range(gs)
                    ]

                def stores(vals, g):
                    for e in range(gs):
                        st[g * gs + e, pl.ds(col, 16)] = vals[e]

                prev = loads(0)
                for g in range(1, hd // gs):
                    cur = loads(g)
                    stores(prev, g - 1)
                    prev = cur
                stores(prev, hd // gs - 1)
                return carry2

            lax.fori_loop(0, nbg, _bg, 0)

        def unit_body(t, carry):
            u = wid * n_units + t
            slab = u // 2
            li = slab // l2
            ji = slab % l2
            d0 = (u % 2) * hd

            @pl.when(jnp.logical_or(t == 0, u % 2 == 0))
            def _():
                pltpu.sync_copy(tm_hbm.at[li, ji], idx_v)
            ktab = kvt_v.at[pl.ds(d0 * vrows, hd * vrows)]
            vtab = kvt_v.at[pl.ds((d + d0) * vrows, hd * vrows)]

            @pl.when(t > 0)
            def _():
                wait_write(kst, k_out.at[0, 0, pl.ds(0, hd)], wsems[0])

            fill(kst, ktab)
            pltpu.async_copy(kst, k_out.at[li, ji, pl.ds(d0, hd)], wsems[0])

            @pl.when(t > 0)
            def _():
                wait_write(vst, v_out.at[0, 0, pl.ds(0, hd)], wsems[1])

            fill(vst, vtab)
            pltpu.async_copy(vst, v_out.at[li, ji, pl.ds(d0, hd)], wsems[1])
            return carry

        lax.fori_loop(0, n_units, unit_body, 0)
        wait_write(kst, k_out.at[0, 0, pl.ds(0, hd)], wsems[0])
        wait_write(vst, v_out.at[0, 0, pl.ds(0, hd)], wsems[1])

    return k(x_flat, midx, didx, tm3, tok_pad, maug, daug, kvt_flat)


def _layernorm_tc(h, gamma, beta, l):
    n, dw = h.shape
    d = gamma.shape[0]
    blk = n // l

    def body(h_ref, g_ref, b_ref, o_ref):
        hv = h_ref[:, :d]
        u = jnp.mean(hv, axis=-1, keepdims=True)
        c = hv - u
        s = jnp.mean(c * c, axis=-1, keepdims=True)
        res = g_ref[...] * (c * lax.rsqrt(s + _EPS)) + b_ref[...]
        o_ref[0] = jnp.transpose(res, (1, 0))

    return pl.pallas_call(
        body,
        grid=(l,),
        in_specs=[
            pl.BlockSpec((blk, dw), lambda i: (i, 0)),
            pl.BlockSpec((1, d), lambda i: (0, 0)),
            pl.BlockSpec((1, d), lambda i: (0, 0)),
        ],
        out_specs=pl.BlockSpec((1, d, blk), lambda i: (i, 0, 0)),
        out_shape=jax.ShapeDtypeStruct((l, d, blk), jnp.float32),
    )(h, gamma.reshape(1, d), beta.reshape(1, d))


def kernel(x, stamp, time_matrix, tok_table, weekday_table, day_table, month_table, K_table, V_table, gamma, beta):
    b, l = x.shape
    d = tok_table.shape[1]
    dw = 2 * d

    x_flat = x.T.reshape(-1)
    tm3 = jnp.transpose(time_matrix, (1, 2, 0))
    vrows = 264
    kvt = jnp.concatenate([K_table.T, V_table.T], axis=0)
    kvt = jnp.pad(kvt, ((0, 0), (0, vrows - kvt.shape[1]))).reshape(-1)
    m_sent = month_table.shape[0]
    d_sent = day_table.shape[0]
    midx = jnp.concatenate(
        [jnp.full((1, b), m_sent, jnp.int32), stamp[:, :, 0].T], axis=0).reshape(-1)
    didx = jnp.concatenate(
        [jnp.full((1, b), d_sent, jnp.int32), stamp[:, :, 1].T], axis=0).reshape(-1)
    tok_pad = jnp.pad(tok_table, ((0, 0), (0, dw - d)))
    maug = jnp.pad(month_table, ((0, 3), (0, dw - d)))
    daug = jnp.pad(day_table, ((0, 8), (0, dw - d)))

    h, kout, vout = _sc_gather(x_flat, midx, didx, tm3, tok_pad,
                               maug, daug, kvt, vrows)
    out = _layernorm_tc(h, gamma, beta, l)
    kf = jnp.transpose(kout, (3, 0, 1, 2))
    vf = jnp.transpose(vout, (3, 0, 1, 2))
    return (jnp.transpose(out, (2, 0, 1)), kf, vf)
